# Initial kernel scaffold; baseline (speedup 1.0000x reference)
import jax
import jax.numpy as jnp
from jax.experimental import pallas as pl


def kernel(sparse_xyz, sparse_feats, nei_inds, sparse_xyz_norm, dense_xyz, dense_xyz_norm, dense_feats, wn_w0, wn_b0, wn_g0, wn_beta0, wn_w1, wn_b1, wn_g1, wn_beta1, wn_w2, wn_b2, wn_g2, wn_beta2, lin_w, lin_b, lin_g, lin_beta):
    raise NotImplementedError("write your pallas kernel here")


# speedup vs baseline: 5.1483x; 5.1483x over previous
import functools

import jax
import jax.numpy as jnp
from jax import lax
from jax.experimental import pallas as pl
from jax.experimental.pallas import tpu as pltpu
from jax.experimental.pallas import tpu_sc as plsc

_M = 50000
_K = 16
_C = 64
_W = 16
_COUT = 64
_MK = _M * _K
_D = 128
_MB = 400
_CH = 1000


def _sc_gather(table, idx):
    info = plsc.get_sparse_core_info()
    nc, ns = info.num_cores, info.num_subcores
    nw = nc * ns
    rows_per = _MK // nw
    nch = rows_per // _CH

    @functools.partial(
        pl.kernel,
        mesh=plsc.VectorSubcoreMesh(core_axis_name="c", subcore_axis_name="s"),
        out_type=jax.ShapeDtypeStruct((_MK, _D), jnp.float32),
        scratch_types=[
            pltpu.VMEM((_CH,), jnp.int32),
            pltpu.VMEM((_CH, _D), jnp.float32),
            pltpu.SemaphoreType.DMA,
        ],
    )
    def gather_kernel(table_hbm, idx_hbm, out_hbm, idx_v, rows_v, sem):
        wid = lax.axis_index("s") * nc + lax.axis_index("c")
        base = wid * rows_per

        def step(i, carry):
            off = base + i * _CH
            pltpu.sync_copy(idx_hbm.at[pl.ds(off, _CH)], idx_v)
            pltpu.async_copy(table_hbm.at[idx_v], rows_v, sem).wait()
            pltpu.sync_copy(rows_v, out_hbm.at[pl.ds(off, _CH)])
            return carry

        lax.fori_loop(0, nch, step, 0)

    return gather_kernel(table, idx)


def _ln_act(x, g, b, act):
    mu = jnp.mean(x, axis=-1, keepdims=True)
    var = jnp.mean((x - mu) * (x - mu), axis=-1, keepdims=True)
    y = (x - mu) * lax.rsqrt(var + 1e-5) * g + b
    if act:
        y = jnp.where(y >= 0, y, 0.1 * y)
    return y


def _tc_body(g_ref, dx_ref, df_ref,
             w0_ref, b0_ref, g0_ref, t0_ref,
             w1_ref, b1_ref, g1_ref, t1_ref,
             w2_ref, b2_ref, g2_ref, t2_ref,
             lw_ref, lb_ref, lg_ref, lt_ref,
             out_ref, loc_ref):
    mbk = _MB * _K
    g = g_ref[...]
    feat = g[:, :_C]
    gx = g[:, _C:_C + 3]
    dx = dx_ref[...]
    drep = jnp.broadcast_to(dx[:, None, :], (_MB, _K, 3)).reshape(mbk, 3)
    loc = gx - drep
    loc_ref[...] = loc

    h = jnp.dot(loc, w0_ref[...], preferred_element_type=jnp.float32)
    h = _ln_act(h + b0_ref[...], g0_ref[...], t0_ref[...], True)
    h = jnp.dot(h, w1_ref[...], preferred_element_type=jnp.float32)
    h = _ln_act(h + b1_ref[...], g1_ref[...], t1_ref[...], True)
    h = jnp.dot(h, w2_ref[...], preferred_element_type=jnp.float32)
    hw = _ln_act(h + b2_ref[...], g2_ref[...], t2_ref[...], False)

    cols = []
    for j in range(_W):
        pj = feat * hw[:, j:j + 1]
        cols.append(jnp.sum(pj.reshape(_MB, _K, _C), axis=1))
    flat = jnp.concatenate(cols, axis=1)

    y = jnp.dot(flat, lw_ref[...], preferred_element_type=jnp.float32)
    y = _ln_act(y + lb_ref[...], lg_ref[...], lt_ref[...], True)
    out_ref[...] = y + df_ref[...]


def _full(shape):
    return pl.BlockSpec(shape, lambda i: (0, 0))


_IN_SPECS = [
    pl.BlockSpec((_MB * _K, _D), lambda i: (i, 0)),
    pl.BlockSpec((_MB, 3), lambda i: (i, 0)),
    pl.BlockSpec((_MB, _COUT), lambda i: (i, 0)),
    _full((3, _W)), _full((1, _W)), _full((1, _W)), _full((1, _W)),
    _full((_W, _W)), _full((1, _W)), _full((1, _W)), _full((1, _W)),
    _full((_W, _W)), _full((1, _W)), _full((1, _W)), _full((1, _W)),
    _full((_C * _W, _COUT)), _full((1, _COUT)), _full((1, _COUT)),
    _full((1, _COUT)),
]
_OUT_SPECS = [
    pl.BlockSpec((_MB, _COUT), lambda i: (i, 0)),
    pl.BlockSpec((_MB * _K, 3), lambda i: (i, 0)),
]
_OUT_SHAPES = [
    jax.ShapeDtypeStruct((_M, _COUT), jnp.float32),
    jax.ShapeDtypeStruct((_MK, 3), jnp.float32),
]


def kernel(sparse_xyz, sparse_feats, nei_inds, sparse_xyz_norm, dense_xyz,
           dense_xyz_norm, dense_feats,
           wn_w0, wn_b0, wn_g0, wn_beta0,
           wn_w1, wn_b1, wn_g1, wn_beta1,
           wn_w2, wn_b2, wn_g2, wn_beta2,
           lin_w, lin_b, lin_g, lin_beta):
    n = sparse_xyz.shape[1]
    table = jnp.concatenate(
        [sparse_feats[0], sparse_xyz[0],
         jnp.zeros((n, _D - _C - 3), jnp.float32)], axis=1)
    idx = nei_inds.reshape(_MK).astype(jnp.int32)
    gathered = _sc_gather(table, idx)

    lw = lin_w.reshape(_C, _W, _COUT).transpose(1, 0, 2).reshape(_C * _W, _COUT)

    operands = (
        gathered, dense_xyz[0], dense_feats[0],
        wn_w0, wn_b0.reshape(1, -1), wn_g0.reshape(1, -1), wn_beta0.reshape(1, -1),
        wn_w1, wn_b1.reshape(1, -1), wn_g1.reshape(1, -1), wn_beta1.reshape(1, -1),
        wn_w2, wn_b2.reshape(1, -1), wn_g2.reshape(1, -1), wn_beta2.reshape(1, -1),
        lw, lin_b.reshape(1, -1), lin_g.reshape(1, -1), lin_beta.reshape(1, -1),
    )
    new_feat, loc = pl.pallas_call(
        _tc_body,
        grid=(_M // _MB,),
        in_specs=_IN_SPECS,
        out_specs=_OUT_SPECS,
        out_shape=_OUT_SHAPES,
    )(*operands)
    return new_feat[None], loc.reshape(1, _M, _K, 3)
